# baseline (speedup 1.0000x reference)
import functools

import jax
import jax.numpy as jnp
from jax import lax
from jax.experimental import pallas as pl
from jax.experimental.pallas import tpu as pltpu
from jax.experimental.pallas import tpu_sc as plsc

_info = plsc.get_sparse_core_info()
_NC, _NS, _L = _info.num_cores, _info.num_subcores, _info.num_lanes
_NW = _NC * _NS

_B = 16384
_PER_W = _B // _NW
_VECS = _PER_W // _L


def _make_kernel():
    mesh = plsc.VectorSubcoreMesh(core_axis_name="c", subcore_axis_name="s")

    @functools.partial(
        pl.kernel,
        mesh=mesh,
        out_type=jax.ShapeDtypeStruct((_B,), jnp.float32),
        scratch_types=[
            pltpu.VMEM((_PER_W,), jnp.int32),
            pltpu.VMEM((_L,), jnp.float32),
            pltpu.VMEM((_PER_W,), jnp.float32),
        ],
    )
    def lookup(idx_hbm, tab_hbm, out_hbm, idx_v, tab_v, out_v):
        wid = lax.axis_index("s") * _NC + lax.axis_index("c")
        base = wid * _PER_W
        pltpu.sync_copy(idx_hbm.at[pl.ds(base, _PER_W)], idx_v)
        pltpu.sync_copy(tab_hbm, tab_v.at[pl.ds(0, 2)])
        tv = tab_v[...]
        t0 = tv[0]
        t1 = tv[1]
        d = t1 - t0
        for i in range(_VECS):
            iv = idx_v[pl.ds(i * _L, _L)]
            fv = iv.astype(jnp.float32)
            out_v[pl.ds(i * _L, _L)] = t0 + fv * d
        pltpu.sync_copy(out_v, out_hbm.at[pl.ds(base, _PER_W)])

    return lookup


_lookup = _make_kernel()


@jax.jit
def kernel(indices, table):
    idx = indices.astype(jnp.int32)
    tab = table.reshape((2,)).astype(jnp.float32)
    out = _lookup(idx, tab)
    return out.reshape((_B, 1))


# speedup vs baseline: 1.0212x; 1.0212x over previous
import functools

import jax
import jax.numpy as jnp
from jax import lax
from jax.experimental import pallas as pl
from jax.experimental.pallas import tpu as pltpu
from jax.experimental.pallas import tpu_sc as plsc

_info = plsc.get_sparse_core_info()
_NC, _NS, _L = _info.num_cores, _info.num_subcores, _info.num_lanes
_NW = _NC * _NS

_B = 16384
_PER_W = _B // _NW
_VECS = _PER_W // _L


def _make_kernel():
    mesh = plsc.VectorSubcoreMesh(core_axis_name="c", subcore_axis_name="s")

    @functools.partial(
        pl.kernel,
        mesh=mesh,
        out_type=jax.ShapeDtypeStruct((_B,), jnp.float32),
        scratch_types=[
            pltpu.VMEM((_PER_W,), jnp.int32),
            pltpu.VMEM((_L,), jnp.float32),
            pltpu.VMEM((_PER_W,), jnp.float32),
            pltpu.SemaphoreType.DMA,
            pltpu.SemaphoreType.DMA,
        ],
    )
    def lookup(idx_hbm, tab_hbm, out_hbm, idx_v, tab_v, out_v, sem_i, sem_t):
        wid = lax.axis_index("s") * _NC + lax.axis_index("c")
        base = wid * _PER_W
        cp_i = pltpu.async_copy(idx_hbm.at[pl.ds(base, _PER_W)], idx_v, sem_i)
        cp_t = pltpu.async_copy(tab_hbm, tab_v.at[pl.ds(0, 2)], sem_t)
        cp_t.wait()
        cp_i.wait()
        tv = tab_v[...]
        t0 = tv[0]
        t1 = tv[1]
        d = t1 - t0
        for i in range(_VECS):
            iv = idx_v[pl.ds(i * _L, _L)]
            fv = iv.astype(jnp.float32)
            out_v[pl.ds(i * _L, _L)] = t0 + fv * d
        pltpu.sync_copy(out_v, out_hbm.at[pl.ds(base, _PER_W)])

    return lookup


_lookup = _make_kernel()


@jax.jit
def kernel(indices, table):
    idx = indices.astype(jnp.int32)
    tab = table.reshape((2,)).astype(jnp.float32)
    out = _lookup(idx, tab)
    return out.reshape((_B, 1))


# speedup vs baseline: 1.1138x; 1.0907x over previous
import functools

import jax
import jax.numpy as jnp
from jax import lax
from jax.experimental import pallas as pl
from jax.experimental.pallas import tpu as pltpu
from jax.experimental.pallas import tpu_sc as plsc

_info = plsc.get_sparse_core_info()
_NC, _NS, _L = 1, _info.num_subcores, _info.num_lanes
_NW = _NC * _NS

_B = 16384
_PER_W = _B // _NW
_VECS = _PER_W // _L


def _make_kernel():
    mesh = plsc.VectorSubcoreMesh(core_axis_name="c", subcore_axis_name="s", num_cores=1)

    @functools.partial(
        pl.kernel,
        mesh=mesh,
        out_type=jax.ShapeDtypeStruct((_B,), jnp.float32),
        scratch_types=[
            pltpu.VMEM((_PER_W,), jnp.int32),
            pltpu.VMEM((_L,), jnp.float32),
            pltpu.VMEM((_PER_W,), jnp.float32),
            pltpu.SemaphoreType.DMA,
            pltpu.SemaphoreType.DMA,
        ],
    )
    def lookup(idx_hbm, tab_hbm, out_hbm, idx_v, tab_v, out_v, sem_i, sem_t):
        wid = lax.axis_index("s") * _NC + lax.axis_index("c")
        base = wid * _PER_W
        cp_i = pltpu.async_copy(idx_hbm.at[pl.ds(base, _PER_W)], idx_v, sem_i)
        cp_t = pltpu.async_copy(tab_hbm, tab_v.at[pl.ds(0, 2)], sem_t)
        cp_t.wait()
        cp_i.wait()
        tv = tab_v[...]
        t0 = tv[0]
        t1 = tv[1]
        d = t1 - t0
        for i in range(_VECS):
            iv = idx_v[pl.ds(i * _L, _L)]
            fv = iv.astype(jnp.float32)
            out_v[pl.ds(i * _L, _L)] = t0 + fv * d
        pltpu.sync_copy(out_v, out_hbm.at[pl.ds(base, _PER_W)])

    return lookup


_lookup = _make_kernel()


@jax.jit
def kernel(indices, table):
    idx = indices.astype(jnp.int32)
    tab = table.reshape((2,)).astype(jnp.float32)
    out = _lookup(idx, tab)
    return out.reshape((_B, 1))
